# Initial kernel scaffold; baseline (speedup 1.0000x reference)
import jax
import jax.numpy as jnp
from jax.experimental import pallas as pl


def kernel(x, edge_index, batch, dropout, W1, b1, W2, b2, W3, b3, Wc1, bc1, Wc2, bc2, Wl, bl, Wl2, bl2, Wl3, bl3, Wl4, bl4):
    raise NotImplementedError("write your pallas kernel here")


# speedup vs baseline: 19.3221x; 19.3221x over previous
import functools

import jax
import jax.numpy as jnp
from jax import lax
from jax.experimental import pallas as pl
from jax.experimental.pallas import tpu as pltpu
from jax.experimental.pallas import tpu_sc as plsc

NC = 2
NS = 16
TR = 2048
KD = 1000
KP = 400
GRAPHS = 64

_HIGH = jax.lax.Precision.HIGHEST

_SC_MESH = plsc.VectorSubcoreMesh(core_axis_name="c", subcore_axis_name="s")
_SC_PARAMS = pltpu.CompilerParams(use_tc_tiling_on_sc=False)


def _dot(a, b):
  return lax.dot_general(a, b, (((1,), (0,)), ((), ())),
                         precision=_HIGH, preferred_element_type=jnp.float32)


def _make_deg(E, NP):
  ept = E // (NC * NS)
  nchunk = ept // KD
  rows_pt = NP // NS

  @functools.partial(
      pl.kernel,
      out_type=[jax.ShapeDtypeStruct((NP,), jnp.float32)] * 2,
      mesh=_SC_MESH,
      compiler_params=_SC_PARAMS,
      scratch_types=[
          pltpu.VMEM((KD,), jnp.int32),
          pltpu.VMEM((KD,), jnp.float32),
          pltpu.VMEM_SHARED((NP,), jnp.float32),
      ],
  )
  def deg_kernel(dst_hbm, ones_hbm, zeros_hbm, deg0_hbm, deg1_hbm,
                 idx_v, ones_v, acc_s):
    cid = lax.axis_index("c")
    sid = lax.axis_index("s")
    pltpu.sync_copy(zeros_hbm, acc_s.at[pl.ds(sid * rows_pt, rows_pt)])
    pltpu.sync_copy(ones_hbm, ones_v)
    plsc.subcore_barrier()
    base = (cid * NS + sid) * ept

    def chunk(k, carry):
      pltpu.sync_copy(dst_hbm.at[pl.ds(base + k * KD, KD)], idx_v)
      pltpu.sync_copy(ones_v, acc_s.at[idx_v], add=True)
      return carry

    lax.fori_loop(0, nchunk, chunk, 0)
    plsc.subcore_barrier()

    @pl.when(cid == 0)
    def _():
      pltpu.sync_copy(acc_s.at[pl.ds(sid * rows_pt, rows_pt)],
                      deg0_hbm.at[pl.ds(sid * rows_pt, rows_pt)])

    @pl.when(cid == 1)
    def _():
      pltpu.sync_copy(acc_s.at[pl.ds(sid * rows_pt, rows_pt)],
                      deg1_hbm.at[pl.ds(sid * rows_pt, rows_pt)])

  return deg_kernel


def _make_prop(E, NP):
  ept = E // NS
  nchunk = ept // KP
  rows_pt = NP // NS

  @functools.partial(
      pl.kernel,
      out_type=[jax.ShapeDtypeStruct((NP, 32), jnp.float32)] * 2,
      mesh=_SC_MESH,
      compiler_params=_SC_PARAMS,
      scratch_types=[
          pltpu.VMEM((KP,), jnp.int32),
          pltpu.VMEM((KP,), jnp.int32),
          pltpu.VMEM((KP, 32), jnp.float32),
          pltpu.VMEM_SHARED((NP, 32), jnp.float32),
          pltpu.SemaphoreType.DMA,
      ],
  )
  def prop_kernel(ht0_hbm, ht1_hbm, src_hbm, dst_hbm, zeros2_hbm,
                  s0_hbm, s1_hbm, src_v, dst_v, rows_v, acc_s, sem):
    cid = lax.axis_index("c")
    sid = lax.axis_index("s")
    pltpu.sync_copy(zeros2_hbm, acc_s.at[pl.ds(sid * rows_pt, rows_pt)])
    plsc.subcore_barrier()
    base = sid * ept

    def run(ht_hbm):
      def chunk(k, carry):
        off = base + k * KP
        pltpu.sync_copy(src_hbm.at[pl.ds(off, KP)], src_v)
        pltpu.sync_copy(dst_hbm.at[pl.ds(off, KP)], dst_v)
        pltpu.async_copy(ht_hbm.at[src_v], rows_v, sem).wait()
        pltpu.sync_copy(rows_v, acc_s.at[dst_v], add=True)
        return carry
      lax.fori_loop(0, nchunk, chunk, 0)

    @pl.when(cid == 0)
    def _():
      run(ht0_hbm)

    @pl.when(cid == 1)
    def _():
      run(ht1_hbm)

    plsc.subcore_barrier()

    @pl.when(cid == 0)
    def _():
      pltpu.sync_copy(acc_s.at[pl.ds(sid * rows_pt, rows_pt)],
                      s0_hbm.at[pl.ds(sid * rows_pt, rows_pt)])

    @pl.when(cid == 1)
    def _():
      pltpu.sync_copy(acc_s.at[pl.ds(sid * rows_pt, rows_pt)],
                      s1_hbm.at[pl.ds(sid * rows_pt, rows_pt)])

  return prop_kernel


def _mlp_body(xT_ref, W1_ref, b1_ref, W2_ref, b2_ref, W3_ref, b3_ref,
              Wc1_ref, out_ref):
  h = lax.dot_general(xT_ref[...], W1_ref[...], (((0,), (0,)), ((), ())),
                      precision=_HIGH, preferred_element_type=jnp.float32)
  h = jnp.maximum(h + b1_ref[...], 0.0)
  h = jnp.maximum(_dot(h, W2_ref[...]) + b2_ref[...], 0.0)
  h = jnp.maximum(_dot(h, W3_ref[...]) + b3_ref[...], 0.0)
  out_ref[...] = _dot(h, Wc1_ref[...])


def _scale_body(p1_ref, d0_ref, d1_ref, ht0_ref, ht1_ref, dis_ref):
  deg = d0_ref[...] + d1_ref[...] + 1.0
  dis = lax.rsqrt(deg)
  ht = p1_ref[...] * dis
  ht0_ref[...] = ht[:, :32]
  ht1_ref[...] = ht[:, 32:]
  dis_ref[...] = dis


def _mid_body(s0_ref, s1_ref, ht0_ref, ht1_ref, dis_ref, b0_ref, b1_ref,
              o0_ref, o1_ref):
  dis = dis_ref[...]
  h4a = jnp.maximum(dis * (s0_ref[...] + ht0_ref[...]) + b0_ref[...], 0.0)
  h4b = jnp.maximum(dis * (s1_ref[...] + ht1_ref[...]) + b1_ref[...], 0.0)
  o0_ref[...] = dis * h4a
  o1_ref[...] = dis * h4b


def _make_segmax(N, NP, NT):
  def body(lo_ref, hi_ref, s0_ref, s1_ref, ht0_ref, ht1_ref, dis_ref,
           batch_ref, Wa_ref, Wb_ref, bc2_ref, out_ref):
    i = pl.program_id(0)

    @pl.when(i == 0)
    def _():
      out_ref[...] = jnp.full(out_ref.shape, -jnp.inf, jnp.float32)

    dis = dis_ref[...]
    a0 = dis * (s0_ref[...] + ht0_ref[...])
    a1 = dis * (s1_ref[...] + ht1_ref[...])
    h5 = jnp.maximum(_dot(a0, Wa_ref[...]) + _dot(a1, Wb_ref[...])
                     + bc2_ref[...], 0.0)
    bv = batch_ref[0]
    rowid = lax.broadcasted_iota(jnp.int32, (TR, 1), 0) + i * TR
    valid = rowid < N
    lo = lo_ref[i]
    hi = hi_ref[i]

    def gbody(g, carry):
      m = (bv == g) & valid
      cur = jnp.max(jnp.where(m, h5, -jnp.inf), axis=0, keepdims=True)
      out_ref[pl.ds(g, 1), :] = jnp.maximum(out_ref[pl.ds(g, 1), :], cur)
      return carry

    lax.fori_loop(lo, hi + 1, gbody, 0)

  return body


def _head_body(g_ref, Wl_ref, bl_ref, Wl2_ref, bl2_ref, Wl3_ref, bl3_ref,
               Wl4_ref, bl4_ref, out_ref):
  g = jnp.maximum(_dot(g_ref[...], Wl_ref[...]) + bl_ref[...], 0.0)
  g = jnp.maximum(_dot(g, Wl2_ref[...]) + bl2_ref[...], 0.0)
  g = jnp.maximum(_dot(g, Wl3_ref[...]) + bl3_ref[...], 0.0)
  out_ref[...] = _dot(g, Wl4_ref[...]) + bl4_ref[...]


def _row_spec(cols):
  return pl.BlockSpec((TR, cols), lambda i: (i, 0))


def _const_spec(shape):
  return pl.BlockSpec(shape, lambda i: tuple(0 for _ in shape))


def kernel(x, edge_index, batch, dropout, W1, b1, W2, b2, W3, b3, Wc1, bc1,
           Wc2, bc2, Wl, bl, Wl2, bl2, Wl3, bl3, Wl4, bl4):
  N = x.shape[0]
  E = edge_index.shape[1]
  NT = -(-N // TR)
  NP = NT * TR
  rows_pt = NP // NS

  src = edge_index[0]
  dst = edge_index[1]
  xT = jnp.pad(x.T, ((0, 0), (0, NP - N)))
  batch_p = jnp.pad(batch, (0, NP - N), constant_values=GRAPHS - 1)
  batch3 = batch_p.reshape(NT, TR, 1)
  tile_lo = batch_p.reshape(NT, TR)[:, 0]
  tile_hi = batch_p.reshape(NT, TR)[:, -1]

  ones_kd = jnp.ones((KD,), jnp.float32)
  zeros_1d = jnp.zeros((rows_pt,), jnp.float32)
  zeros_2d = jnp.zeros((rows_pt, 32), jnp.float32)

  deg0, deg1 = _make_deg(E, NP)(dst, ones_kd, zeros_1d)
  deg0 = deg0.reshape(NP, 1)
  deg1 = deg1.reshape(NP, 1)

  p1 = pl.pallas_call(
      _mlp_body,
      grid=(NT,),
      in_specs=[
          pl.BlockSpec((2, TR), lambda i: (0, i)),
          _const_spec((2, 64)), _const_spec((1, 64)),
          _const_spec((64, 128)), _const_spec((1, 128)),
          _const_spec((128, 512)), _const_spec((1, 512)),
          _const_spec((512, 64)),
      ],
      out_specs=_row_spec(64),
      out_shape=jax.ShapeDtypeStruct((NP, 64), jnp.float32),
  )(xT, W1, b1.reshape(1, 64), W2, b2.reshape(1, 128),
    W3, b3.reshape(1, 512), Wc1)

  ht0, ht1, dis = pl.pallas_call(
      _scale_body,
      grid=(NT,),
      in_specs=[_row_spec(64), _row_spec(1), _row_spec(1)],
      out_specs=[_row_spec(32), _row_spec(32), _row_spec(1)],
      out_shape=[
          jax.ShapeDtypeStruct((NP, 32), jnp.float32),
          jax.ShapeDtypeStruct((NP, 32), jnp.float32),
          jax.ShapeDtypeStruct((NP, 1), jnp.float32),
      ],
  )(p1, deg0, deg1)

  prop = _make_prop(E, NP)

  s0, s1 = prop(ht0, ht1, src, dst, zeros_2d)

  ht20, ht21 = pl.pallas_call(
      _mid_body,
      grid=(NT,),
      in_specs=[_row_spec(32), _row_spec(32), _row_spec(32), _row_spec(32),
                _row_spec(1), _const_spec((1, 32)), _const_spec((1, 32))],
      out_specs=[_row_spec(32), _row_spec(32)],
      out_shape=[
          jax.ShapeDtypeStruct((NP, 32), jnp.float32),
          jax.ShapeDtypeStruct((NP, 32), jnp.float32),
      ],
  )(s0, s1, ht0, ht1, dis, bc1[:32].reshape(1, 32), bc1[32:].reshape(1, 32))

  s20, s21 = prop(ht20, ht21, src, dst, zeros_2d)

  gmax = pl.pallas_call(
      _make_segmax(N, NP, NT),
      grid_spec=pltpu.PrefetchScalarGridSpec(
          num_scalar_prefetch=2,
          grid=(NT,),
          in_specs=[
              pl.BlockSpec((TR, 32), lambda i, lo, hi: (i, 0)),
              pl.BlockSpec((TR, 32), lambda i, lo, hi: (i, 0)),
              pl.BlockSpec((TR, 32), lambda i, lo, hi: (i, 0)),
              pl.BlockSpec((TR, 32), lambda i, lo, hi: (i, 0)),
              pl.BlockSpec((TR, 1), lambda i, lo, hi: (i, 0)),
              pl.BlockSpec((1, TR, 1), lambda i, lo, hi: (i, 0, 0)),
              pl.BlockSpec((32, 256), lambda i, lo, hi: (0, 0)),
              pl.BlockSpec((32, 256), lambda i, lo, hi: (0, 0)),
              pl.BlockSpec((1, 256), lambda i, lo, hi: (0, 0)),
          ],
          out_specs=pl.BlockSpec((GRAPHS, 256), lambda i, lo, hi: (0, 0)),
      ),
      out_shape=jax.ShapeDtypeStruct((GRAPHS, 256), jnp.float32),
  )(tile_lo, tile_hi, s20, s21, ht20, ht21, dis, batch3,
    Wc2[:32], Wc2[32:], bc2.reshape(1, 256))

  out = pl.pallas_call(
      _head_body,
      in_specs=[pl.BlockSpec((GRAPHS, 256), lambda: (0, 0))] + [
          pl.BlockSpec(s, lambda: (0, 0)) for s in
          [(256, 128), (1, 128), (128, 64), (1, 64),
           (64, 28), (1, 28), (28, 2), (1, 2)]
      ],
      out_specs=pl.BlockSpec((GRAPHS, 2), lambda: (0, 0)),
      out_shape=jax.ShapeDtypeStruct((GRAPHS, 2), jnp.float32),
  )(gmax, Wl, bl.reshape(1, 128), Wl2, bl2.reshape(1, 64),
    Wl3, bl3.reshape(1, 28), Wl4, bl4.reshape(1, 2))

  return out
